# Initial kernel scaffold; baseline (speedup 1.0000x reference)
import jax
import jax.numpy as jnp
from jax.experimental import pallas as pl


def kernel(node_embedding, pos, atomic_numbers, batch, natoms, W1, b1, W2, b2):
    raise NotImplementedError("write your pallas kernel here")


# speedup vs baseline: 1.8942x; 1.8942x over previous
import functools

import jax
import jax.numpy as jnp
from jax import lax
from jax.experimental import pallas as pl
from jax.experimental.pallas import tpu as pltpu
from jax.experimental.pallas import tpu_sc as plsc


def _mlp_body(x_ref, w1t_ref, b1_ref, w2_ref, b2_ref, out_ref):
    x = x_ref[...]
    w1t = w1t_ref[...]
    b1 = b1_ref[...]
    h = jnp.dot(x, w1t, preferred_element_type=jnp.float32) + b1
    h = h * jax.nn.sigmoid(h)
    h = jnp.dot(h, w1t, preferred_element_type=jnp.float32) + b1
    h = h * jax.nn.sigmoid(h)
    pred = jnp.dot(h, w2_ref[...], preferred_element_type=jnp.float32)
    out_ref[...] = pred + b2_ref[...]


def _mlp_pred(node_embedding, W1, b1, W2, b2, block_rows):
    n, d = node_embedding.shape
    grid = n // block_rows
    return pl.pallas_call(
        _mlp_body,
        grid=(grid,),
        in_specs=[
            pl.BlockSpec((block_rows, d), lambda i: (i, 0)),
            pl.BlockSpec((d, d), lambda i: (0, 0)),
            pl.BlockSpec((1, d), lambda i: (0, 0)),
            pl.BlockSpec((d, 1), lambda i: (0, 0)),
            pl.BlockSpec((1, 1), lambda i: (0, 0)),
        ],
        out_specs=pl.BlockSpec((block_rows, 1), lambda i: (i, 0)),
        out_shape=jax.ShapeDtypeStruct((n, 1), jnp.float32),
    )(node_embedding, W1.T, b1.reshape(1, d), W2.T, b2.reshape(1, 1))


_NSUB = 16
_LANE = 128


def _make_seg_sum(nj, s_pad):
    mesh = plsc.VectorSubcoreMesh(core_axis_name="c", subcore_axis_name="s")

    @functools.partial(
        pl.kernel,
        out_type=jax.ShapeDtypeStruct((s_pad,), jnp.float32),
        mesh=mesh,
        scratch_types=[
            pltpu.VMEM((nj, _LANE), jnp.float32),
            pltpu.VMEM((nj, _LANE), jnp.int32),
            pltpu.VMEM((s_pad,), jnp.float32),
            pltpu.VMEM_SHARED((s_pad,), jnp.float32),
        ],
    )
    def seg_sum(pred_hbm, ids_hbm, out_hbm, vals_v, ids_v, zero_v, acc_sh):
        cid = lax.axis_index("c")
        sid = lax.axis_index("s")

        @pl.when(cid == 0)
        def _():
            pltpu.sync_copy(pred_hbm.at[sid], vals_v)
            pltpu.sync_copy(ids_hbm.at[sid], ids_v)

            @pl.when(sid == 0)
            def _zero():
                @pl.loop(0, s_pad, step=16)
                def _(i):
                    zero_v[pl.ds(i, 16)] = jnp.zeros((16,), jnp.float32)

                pltpu.sync_copy(zero_v, acc_sh)

            plsc.subcore_barrier()

            @pl.loop(0, nj)
            def _(j):
                pltpu.sync_copy(vals_v.at[j], acc_sh.at[ids_v.at[j]], add=True)

            plsc.subcore_barrier()

            @pl.when(sid == 0)
            def _out():
                pltpu.sync_copy(acc_sh, out_hbm)

    return seg_sum


def kernel(node_embedding, pos, atomic_numbers, batch, natoms, W1, b1, W2, b2):
    n, d = node_embedding.shape
    s = natoms.shape[0]

    block_rows = 5000
    pred = _mlp_pred(node_embedding, W1, b1, W2, b2, block_rows)

    nj = -(-n // (_NSUB * _LANE))
    n_pad = _NSUB * nj * _LANE
    s_pad = -(-s // 128) * 128

    pred_flat = jnp.pad(pred[:, 0], (0, n_pad - n))
    ids_flat = jnp.pad(batch, (0, n_pad - n))

    pred3 = pred_flat.reshape(_NSUB, nj, _LANE)
    ids3 = ids_flat.reshape(_NSUB, nj, _LANE)

    energy = _make_seg_sum(nj, s_pad)(pred3, ids3)
    return energy[:s]
